# baseline (speedup 1.0000x reference)
import functools

import numpy as np

import jax
import jax.numpy as jnp
from jax import lax
from jax.experimental import pallas as pl
from jax.experimental.pallas import tpu as pltpu
from jax.experimental.pallas import tpu_sc as plsc

N = 10000
E = 320000
NG = 64
HD = 128
EF = 16
MAXZ = 100
CUT = 10.0
RBOUND = 10.0
LAYERS = 4
C = 1.0
Y_STD = 1.0
Y_MEAN = 0.0

NC = 2
NS = 16
NW = NC * NS
EPT = E // NW
K = 80
NCH = EPT // K
K2 = 40
NCH2 = EPT // K2
SPT = N // NS

_mesh = plsc.VectorSubcoreMesh(core_axis_name="c", subcore_axis_name="s")
_sc_params = pltpu.CompilerParams(use_tc_tiling_on_sc=False,
                                  needs_layout_passes=False)

_PERM_LO = np.concatenate([np.arange(32 * k, 32 * k + 16) for k in range(HD // 32)])
_PERM_HI = _PERM_LO + 16


def _silu(x):
    return x * (1.0 / (1.0 + jnp.exp(-x)))


@functools.partial(
    pl.kernel,
    out_type=[
        jax.ShapeDtypeStruct((E, 16), jnp.float32),
        jax.ShapeDtypeStruct((E, 16), jnp.float32),
    ],
    mesh=_mesh,
    scratch_types=[
        pltpu.VMEM((NCH, K), jnp.int32),
        pltpu.VMEM((NCH, K), jnp.int32),
        pltpu.VMEM((K, 16), jnp.float32),
        pltpu.VMEM((K, 16), jnp.float32),
        pltpu.SemaphoreType.DMA,
        pltpu.SemaphoreType.DMA,
    ],
    compiler_params=_sc_params,
)
def _sc_geom(pos_hbm, src_hbm, dst_hbm, ps_hbm, pd_hbm,
             src_v, dst_v, ps_v, pd_v, sem_a, sem_b):
    cid = lax.axis_index("c")
    sid = lax.axis_index("s")
    wid = cid * NS + sid
    base = wid * EPT
    pltpu.sync_copy(src_hbm.at[wid], src_v)
    pltpu.sync_copy(dst_hbm.at[wid], dst_v)

    @pl.loop(0, NCH)
    def _chunk(j):
        ca = pltpu.async_copy(pos_hbm.at[src_v.at[j]], ps_v, sem_a)
        cb = pltpu.async_copy(pos_hbm.at[dst_v.at[j]], pd_v, sem_b)
        ca.wait()
        cb.wait()
        pltpu.sync_copy(ps_v, ps_hbm.at[pl.ds(base + j * K, K)])
        pltpu.sync_copy(pd_v, pd_hbm.at[pl.ds(base + j * K, K)])


@functools.partial(
    pl.kernel,
    out_type=jax.ShapeDtypeStruct((NC, N, HD), jnp.float32),
    mesh=_mesh,
    scratch_types=[
        pltpu.VMEM((NCH2, K2), jnp.int32),
        pltpu.VMEM((NCH2, K2), jnp.int32),
        pltpu.VMEM((K2, HD), jnp.float32),
        pltpu.VMEM((K2, HD), jnp.float32),
        pltpu.VMEM((K2, HD // 2), jnp.float32),
        pltpu.VMEM((K2, HD // 2), jnp.float32),
        pltpu.VMEM_SHARED((N, HD), jnp.float32),
        pltpu.SemaphoreType.DMA,
        pltpu.SemaphoreType.DMA,
        pltpu.SemaphoreType.DMA,
        pltpu.SemaphoreType.DMA,
    ],
    compiler_params=_sc_params,
)
def _sc_msg(msg_hbm, gate_hbm, src_hbm, dst_hbm, zeros_hbm, agg2_hbm,
            src_v, dst_v, rows_v0, rows_v1, gate_v0, gate_v1, agg_sh,
            sem_r0, sem_r1, sem_g0, sem_g1):
    rows = (rows_v0, rows_v1)
    gbuf = (gate_v0, gate_v1)
    sem_r = (sem_r0, sem_r1)
    sem_g = (sem_g0, sem_g1)
    cid = lax.axis_index("c")
    sid = lax.axis_index("s")
    wid = cid * NS + sid
    base = wid * EPT
    pltpu.sync_copy(zeros_hbm.at[pl.ds(sid * SPT, SPT)],
                    agg_sh.at[pl.ds(sid * SPT, SPT)])
    pltpu.sync_copy(src_hbm.at[wid], src_v)
    pltpu.sync_copy(dst_hbm.at[wid], dst_v)
    plsc.subcore_barrier()

    def issue(j, b):
        pltpu.async_copy(gate_hbm.at[pl.ds(base + j * K2, K2)], gbuf[b], sem_g[b])
        pltpu.async_copy(msg_hbm.at[src_v.at[j]], rows[b], sem_r[b])

    def work(j, b):
        pltpu.make_async_copy(gate_hbm.at[pl.ds(base + j * K2, K2)],
                              gbuf[b], sem_g[b]).wait()
        pltpu.make_async_copy(msg_hbm.at[src_v.at[j]], rows[b], sem_r[b]).wait()

        @pl.loop(0, K2)
        def _row(i):
            for k in range(HD // 32):
                v = plsc.bitcast(gbuf[b][i, pl.ds(k * 16, 16)], jnp.uint32)
                ga = plsc.bitcast(v << 16, jnp.float32)
                gb2 = plsc.bitcast(v & jnp.uint32(0xFFFF0000), jnp.float32)
                sl_a = pl.ds(k * 32, 16)
                sl_b = pl.ds(k * 32 + 16, 16)
                rows[b][i, sl_a] = rows[b][i, sl_a] * ga
                rows[b][i, sl_b] = rows[b][i, sl_b] * gb2

        pltpu.sync_copy(rows[b], agg_sh.at[dst_v.at[j]], add=True)

    issue(0, 0)
    issue(1, 1)

    @pl.loop(0, (NCH2 - 2) // 2)
    def _pair(p):
        j0 = p * 2
        work(j0, 0)
        issue(j0 + 2, 0)
        work(j0 + 1, 1)
        issue(j0 + 3, 1)

    work(NCH2 - 2, 0)
    work(NCH2 - 1, 1)

    plsc.subcore_barrier()
    pltpu.sync_copy(agg_sh.at[pl.ds(sid * SPT, SPT)],
                    agg2_hbm.at[cid, pl.ds(sid * SPT, SPT)])


EB = 8000


def _tc_d2_body(ps_ref, pd_ref, out_ref):
    diff = ps_ref[...] - pd_ref[...]
    out_ref[...] = jnp.sum(diff * diff, axis=1, keepdims=True)


def _tc_d2(ps, pd):
    return pl.pallas_call(
        _tc_d2_body,
        grid=(E // EB,),
        in_specs=[
            pl.BlockSpec((EB, 16), lambda j: (j, 0)),
            pl.BlockSpec((EB, 16), lambda j: (j, 0)),
        ],
        out_specs=pl.BlockSpec((EB, 1), lambda j: (j, 0)),
        out_shape=jax.ShapeDtypeStruct((E, 1), jnp.float32),
    )(ps, pd)


def _tc_gates_body(d2_ref, wlo_ref, whi_ref, out_ref):
    d2 = d2_ref[:, 0]
    dist = jnp.sqrt(d2 + 1e-12)
    step = RBOUND / (EF - 1)
    centers = lax.broadcasted_iota(jnp.int32, (EB, EF), 1).astype(jnp.float32) * step
    gamma = (EF / RBOUND) ** 2
    ef = jnp.exp(-gamma * (dist[:, None] - centers) ** 2)
    cut = 0.5 * (jnp.cos(jnp.pi * jnp.clip(dist / CUT, 0.0, 1.0)) + 1.0)
    cut = (cut * (dist < CUT).astype(jnp.float32))[:, None]
    g_lo = _silu(jnp.dot(ef, wlo_ref[...], preferred_element_type=jnp.float32,
                         precision=lax.Precision.HIGHEST)) * cut
    g_hi = _silu(jnp.dot(ef, whi_ref[...], preferred_element_type=jnp.float32,
                         precision=lax.Precision.HIGHEST)) * cut
    lo = lax.bitcast_convert_type(g_lo.astype(jnp.bfloat16),
                                  jnp.uint16).astype(jnp.uint32)
    hi = lax.bitcast_convert_type(g_hi.astype(jnp.bfloat16),
                                  jnp.uint16).astype(jnp.uint32)
    out_ref[...] = lax.bitcast_convert_type(lo | (hi << 16), jnp.float32)


def _tc_gates(d2, w_lo, w_hi):
    return pl.pallas_call(
        _tc_gates_body,
        grid=(E // EB,),
        in_specs=[
            pl.BlockSpec((EB, 1), lambda j: (j, 0)),
            pl.BlockSpec((EF, HD // 2), lambda j: (0, 0)),
            pl.BlockSpec((EF, HD // 2), lambda j: (0, 0)),
        ],
        out_specs=pl.BlockSpec((EB, HD // 2), lambda j: (j, 0)),
        out_shape=jax.ShapeDtypeStruct((E, HD // 2), jnp.float32),
    )(d2, w_lo, w_hi)


NB = 1000


def _tc_pre_body(z_ref, emb_ref, wm_ref, s_ref, m_ref):
    zb = z_ref[...]
    oh = (zb == lax.broadcasted_iota(jnp.int32, (NB, HD), 1)).astype(jnp.float32)
    s = jnp.dot(oh, emb_ref[...], preferred_element_type=jnp.float32, precision=lax.Precision.HIGHEST)
    s_ref[...] = s
    m_ref[...] = jnp.dot(s, wm_ref[...], preferred_element_type=jnp.float32, precision=lax.Precision.HIGHEST)


def _tc_pre(z2, emb_pad, wm0):
    return pl.pallas_call(
        _tc_pre_body,
        grid=(N // NB,),
        in_specs=[
            pl.BlockSpec((NB, 1), lambda j: (j, 0)),
            pl.BlockSpec((HD, HD), lambda j: (0, 0)),
            pl.BlockSpec((HD, HD), lambda j: (0, 0)),
        ],
        out_specs=[
            pl.BlockSpec((NB, HD), lambda j: (j, 0)),
            pl.BlockSpec((NB, HD), lambda j: (j, 0)),
        ],
        out_shape=[
            jax.ShapeDtypeStruct((N, HD), jnp.float32),
            jax.ShapeDtypeStruct((N, HD), jnp.float32),
        ],
    )(z2, emb_pad, wm0)


def _tc_upd_body(s_ref, a0_ref, a1_ref, wus_ref, wua_ref, b_ref, wm_ref,
                 s2_ref, m_ref):
    agg = a0_ref[0] + a1_ref[0]
    pre = (jnp.dot(s_ref[...], wus_ref[...], preferred_element_type=jnp.float32, precision=lax.Precision.HIGHEST)
           + jnp.dot(agg, wua_ref[...], preferred_element_type=jnp.float32, precision=lax.Precision.HIGHEST)
           + b_ref[...])
    s2 = s_ref[...] + _silu(pre)
    s2_ref[...] = s2
    m_ref[...] = jnp.dot(s2, wm_ref[...], preferred_element_type=jnp.float32, precision=lax.Precision.HIGHEST)


def _tc_upd(scalar, agg2, wus, wua, b, wm_next):
    return pl.pallas_call(
        _tc_upd_body,
        grid=(N // NB,),
        in_specs=[
            pl.BlockSpec((NB, HD), lambda j: (j, 0)),
            pl.BlockSpec((1, NB, HD), lambda j: (0, j, 0)),
            pl.BlockSpec((1, NB, HD), lambda j: (1, j, 0)),
            pl.BlockSpec((HD, HD), lambda j: (0, 0)),
            pl.BlockSpec((HD, HD), lambda j: (0, 0)),
            pl.BlockSpec((1, HD), lambda j: (0, 0)),
            pl.BlockSpec((HD, HD), lambda j: (0, 0)),
        ],
        out_specs=[
            pl.BlockSpec((NB, HD), lambda j: (j, 0)),
            pl.BlockSpec((NB, HD), lambda j: (j, 0)),
        ],
        out_shape=[
            jax.ShapeDtypeStruct((N, HD), jnp.float32),
            jax.ShapeDtypeStruct((N, HD), jnp.float32),
        ],
    )(scalar, agg2, agg2, wus, wua, b, wm_next)


def _tc_final_body(s_ref, a0_ref, a1_ref, wus_ref, wua_ref, b_ref, bi_ref,
                   wo_ref, acc_ref, pred_ref):
    j = pl.program_id(0)

    @pl.when(j == 0)
    def _():
        acc_ref[...] = jnp.zeros((NG, HD), jnp.float32)

    agg = a0_ref[0] + a1_ref[0]
    pre = (jnp.dot(s_ref[...], wus_ref[...], preferred_element_type=jnp.float32, precision=lax.Precision.HIGHEST)
           + jnp.dot(agg, wua_ref[...], preferred_element_type=jnp.float32, precision=lax.Precision.HIGHEST)
           + b_ref[...])
    s2 = s_ref[...] + _silu(pre)
    oh = (bi_ref[...] == lax.broadcasted_iota(jnp.int32, (NB, NG), 1)).astype(jnp.float32)
    acc_ref[...] += lax.dot_general(oh, s2, (((0,), (0,)), ((), ())),
                                    preferred_element_type=jnp.float32,
                                    precision=lax.Precision.HIGHEST)

    @pl.when(j == pl.num_programs(0) - 1)
    def _():
        graph = acc_ref[...] * C
        pred = jnp.sum(graph * wo_ref[...], axis=1, keepdims=True)
        pred_ref[...] = pred * Y_STD + Y_MEAN


def _tc_final(scalar, agg2, wus, wua, b, bi2, wo_t):
    return pl.pallas_call(
        _tc_final_body,
        grid=(N // NB,),
        in_specs=[
            pl.BlockSpec((NB, HD), lambda j: (j, 0)),
            pl.BlockSpec((1, NB, HD), lambda j: (0, j, 0)),
            pl.BlockSpec((1, NB, HD), lambda j: (1, j, 0)),
            pl.BlockSpec((HD, HD), lambda j: (0, 0)),
            pl.BlockSpec((HD, HD), lambda j: (0, 0)),
            pl.BlockSpec((1, HD), lambda j: (0, 0)),
            pl.BlockSpec((NB, 1), lambda j: (j, 0)),
            pl.BlockSpec((1, HD), lambda j: (0, 0)),
        ],
        out_specs=[
            pl.BlockSpec((NG, HD), lambda j: (0, 0)),
            pl.BlockSpec((NG, 1), lambda j: (0, 0)),
        ],
        out_shape=[
            jax.ShapeDtypeStruct((NG, HD), jnp.float32),
            jax.ShapeDtypeStruct((NG, 1), jnp.float32),
        ],
    )(scalar, agg2, agg2, wus, wua, b, bi2, wo_t)


def kernel(pos, z, x, edge_index, batch_index, subg_node_index,
           subg_node_center_index, subg_edge_index, subg_batch_index,
           subg_node_label, emb_table, W_ef, W_msg, W_upd, b_upd, W_out):
    src = edge_index[0].astype(jnp.int32).reshape(NW, NCH, K)
    dst = edge_index[1].astype(jnp.int32).reshape(NW, NCH, K)
    src2 = edge_index[0].astype(jnp.int32).reshape(NW, NCH2, K2)
    dst2 = edge_index[1].astype(jnp.int32).reshape(NW, NCH2, K2)
    zeros_nh = jnp.zeros((N, HD), jnp.float32)
    emb_pad = jnp.zeros((HD, HD), jnp.float32).at[:MAXZ].set(emb_table)

    pos16 = jnp.pad(pos.astype(jnp.float32), ((0, 0), (0, 13)))
    ps, pd = _sc_geom(pos16, src, dst)
    d2 = _tc_d2(ps, pd)
    gates = [_tc_gates(d2, W_ef[l][:, _PERM_LO], W_ef[l][:, _PERM_HI])
             for l in range(LAYERS)]

    scalar, node_msg = _tc_pre(z.astype(jnp.int32).reshape(N, 1), emb_pad, W_msg[0])

    for l in range(LAYERS):
        agg2 = _sc_msg(node_msg, gates[l], src2, dst2, zeros_nh)
        wus = W_upd[l, :HD]
        wua = W_upd[l, HD:]
        b = b_upd[l].reshape(1, HD)
        if l < LAYERS - 1:
            scalar, node_msg = _tc_upd(scalar, agg2, wus, wua, b, W_msg[l + 1])
        else:
            _, pred = _tc_final(scalar, agg2, wus, wua, b,
                                batch_index.astype(jnp.int32).reshape(N, 1),
                                W_out.reshape(1, HD))
    return pred


# speedup vs baseline: 1.8692x; 1.8692x over previous
import functools

import numpy as np

import jax
import jax.numpy as jnp
from jax import lax
from jax.experimental import pallas as pl
from jax.experimental.pallas import tpu as pltpu
from jax.experimental.pallas import tpu_sc as plsc

N = 10000
E = 320000
NG = 64
HD = 128
EF = 16
MAXZ = 100
CUT = 10.0
RBOUND = 10.0
LAYERS = 4
C = 1.0
Y_STD = 1.0
Y_MEAN = 0.0

NC = 2
NS = 16
NW = NC * NS
EPT = E // NW
K = 80
NCH = EPT // K
K2 = 40
NCH2 = EPT // K2
SPT = N // NS

_mesh = plsc.VectorSubcoreMesh(core_axis_name="c", subcore_axis_name="s")
_sc_params = pltpu.CompilerParams(use_tc_tiling_on_sc=False)

_PERM_LO = np.concatenate([np.arange(32 * k, 32 * k + 16) for k in range(HD // 32)])
_PERM_HI = _PERM_LO + 16


def _silu(x):
    return x * (1.0 / (1.0 + jnp.exp(-x)))


@functools.partial(
    pl.kernel,
    out_type=[
        jax.ShapeDtypeStruct((E, 16), jnp.float32),
        jax.ShapeDtypeStruct((E, 16), jnp.float32),
    ],
    mesh=_mesh,
    scratch_types=[
        pltpu.VMEM((NCH, K), jnp.int32),
        pltpu.VMEM((NCH, K), jnp.int32),
        pltpu.VMEM((K, 16), jnp.float32),
        pltpu.VMEM((K, 16), jnp.float32),
        pltpu.SemaphoreType.DMA,
        pltpu.SemaphoreType.DMA,
    ],
    compiler_params=_sc_params,
)
def _sc_geom(pos_hbm, src_hbm, dst_hbm, ps_hbm, pd_hbm,
             src_v, dst_v, ps_v, pd_v, sem_a, sem_b):
    cid = lax.axis_index("c")
    sid = lax.axis_index("s")
    wid = cid * NS + sid
    base = wid * EPT
    pltpu.sync_copy(src_hbm.at[wid], src_v)
    pltpu.sync_copy(dst_hbm.at[wid], dst_v)

    @pl.loop(0, NCH)
    def _chunk(j):
        ca = pltpu.async_copy(pos_hbm.at[src_v.at[j]], ps_v, sem_a)
        cb = pltpu.async_copy(pos_hbm.at[dst_v.at[j]], pd_v, sem_b)
        ca.wait()
        cb.wait()
        pltpu.sync_copy(ps_v, ps_hbm.at[pl.ds(base + j * K, K)])
        pltpu.sync_copy(pd_v, pd_hbm.at[pl.ds(base + j * K, K)])


@functools.partial(
    pl.kernel,
    out_type=jax.ShapeDtypeStruct((NC, N, HD), jnp.float32),
    mesh=_mesh,
    scratch_types=[
        pltpu.VMEM((NCH2, K2), jnp.int32),
        pltpu.VMEM((NCH2, K2), jnp.int32),
        pltpu.VMEM((K2, HD), jnp.float32),
        pltpu.VMEM((K2, HD), jnp.float32),
        pltpu.VMEM((K2, HD // 2), jnp.float32),
        pltpu.VMEM((K2, HD // 2), jnp.float32),
        pltpu.VMEM_SHARED((N, HD), jnp.float32),
        pltpu.SemaphoreType.DMA,
        pltpu.SemaphoreType.DMA,
        pltpu.SemaphoreType.DMA,
        pltpu.SemaphoreType.DMA,
    ],
    compiler_params=_sc_params,
)
def _sc_msg(msg_hbm, gate_hbm, src_hbm, dst_hbm, zeros_hbm, agg2_hbm,
            src_v, dst_v, rows_v0, rows_v1, gate_v0, gate_v1, agg_sh,
            sem_r0, sem_r1, sem_g0, sem_g1):
    rows = (rows_v0, rows_v1)
    gbuf = (gate_v0, gate_v1)
    sem_r = (sem_r0, sem_r1)
    sem_g = (sem_g0, sem_g1)
    cid = lax.axis_index("c")
    sid = lax.axis_index("s")
    wid = cid * NS + sid
    base = wid * EPT
    pltpu.sync_copy(zeros_hbm.at[pl.ds(sid * SPT, SPT)],
                    agg_sh.at[pl.ds(sid * SPT, SPT)])
    pltpu.sync_copy(src_hbm.at[wid], src_v)
    pltpu.sync_copy(dst_hbm.at[wid], dst_v)
    plsc.subcore_barrier()

    def issue(j, b):
        pltpu.async_copy(gate_hbm.at[pl.ds(base + j * K2, K2)], gbuf[b], sem_g[b])
        pltpu.async_copy(msg_hbm.at[src_v.at[j]], rows[b], sem_r[b])

    def work(j, b):
        pltpu.make_async_copy(gate_hbm.at[pl.ds(base + j * K2, K2)],
                              gbuf[b], sem_g[b]).wait()
        pltpu.make_async_copy(msg_hbm.at[src_v.at[j]], rows[b], sem_r[b]).wait()

        @pl.loop(0, K2)
        def _row(i):
            for k in range(HD // 32):
                v = lax.bitcast_convert_type(gbuf[b][i, pl.ds(k * 16, 16)],
                                             jnp.uint32)
                ga = lax.bitcast_convert_type(v << 16, jnp.float32)
                gb2 = lax.bitcast_convert_type(v & jnp.uint32(0xFFFF0000),
                                               jnp.float32)
                sl_a = pl.ds(k * 32, 16)
                sl_b = pl.ds(k * 32 + 16, 16)
                rows[b][i, sl_a] = rows[b][i, sl_a] * ga
                rows[b][i, sl_b] = rows[b][i, sl_b] * gb2

        pltpu.sync_copy(rows[b], agg_sh.at[dst_v.at[j]], add=True)

    issue(0, 0)
    issue(1, 1)

    @pl.loop(0, (NCH2 - 2) // 2)
    def _pair(p):
        j0 = p * 2
        work(j0, 0)
        issue(j0 + 2, 0)
        work(j0 + 1, 1)
        issue(j0 + 3, 1)

    work(NCH2 - 2, 0)
    work(NCH2 - 1, 1)

    plsc.subcore_barrier()
    pltpu.sync_copy(agg_sh.at[pl.ds(sid * SPT, SPT)],
                    agg2_hbm.at[cid, pl.ds(sid * SPT, SPT)])


EB = 8000


def _tc_ef_body(ps_ref, pd_ref, ef_ref, cut_ref):
    diff = ps_ref[...] - pd_ref[...]
    d2 = jnp.sum(diff * diff, axis=1)
    dist = jnp.sqrt(d2 + 1e-12)
    step = RBOUND / (EF - 1)
    centers = lax.broadcasted_iota(jnp.int32, (EB, EF), 1).astype(jnp.float32) * step
    gamma = (EF / RBOUND) ** 2
    ef_ref[...] = jnp.exp(-gamma * (dist[:, None] - centers) ** 2)
    cut = 0.5 * (jnp.cos(jnp.pi * jnp.clip(dist / CUT, 0.0, 1.0)) + 1.0)
    cut_ref[...] = (cut * (dist < CUT).astype(jnp.float32))[:, None]


def _tc_ef(ps, pd):
    return pl.pallas_call(
        _tc_ef_body,
        grid=(E // EB,),
        in_specs=[
            pl.BlockSpec((EB, 16), lambda j: (j, 0)),
            pl.BlockSpec((EB, 16), lambda j: (j, 0)),
        ],
        out_specs=[
            pl.BlockSpec((EB, EF), lambda j: (j, 0)),
            pl.BlockSpec((EB, 1), lambda j: (j, 0)),
        ],
        out_shape=[
            jax.ShapeDtypeStruct((E, EF), jnp.float32),
            jax.ShapeDtypeStruct((E, 1), jnp.float32),
        ],
    )(ps, pd)


def _tc_gates_body(ef_ref, cut_ref, wlo_ref, whi_ref, out_ref):
    ef = ef_ref[...]
    cut = cut_ref[...]
    g_lo = _silu(jnp.dot(ef, wlo_ref[...], preferred_element_type=jnp.float32)) * cut
    g_hi = _silu(jnp.dot(ef, whi_ref[...], preferred_element_type=jnp.float32)) * cut
    lo = lax.bitcast_convert_type(g_lo.astype(jnp.bfloat16),
                                  jnp.uint16).astype(jnp.uint32)
    hi = lax.bitcast_convert_type(g_hi.astype(jnp.bfloat16),
                                  jnp.uint16).astype(jnp.uint32)
    out_ref[...] = lax.bitcast_convert_type(lo | (hi << 16), jnp.float32)


def _tc_gates(ef, cut, w_lo, w_hi):
    return pl.pallas_call(
        _tc_gates_body,
        grid=(E // EB,),
        in_specs=[
            pl.BlockSpec((EB, EF), lambda j: (j, 0)),
            pl.BlockSpec((EB, 1), lambda j: (j, 0)),
            pl.BlockSpec((EF, HD // 2), lambda j: (0, 0)),
            pl.BlockSpec((EF, HD // 2), lambda j: (0, 0)),
        ],
        out_specs=pl.BlockSpec((EB, HD // 2), lambda j: (j, 0)),
        out_shape=jax.ShapeDtypeStruct((E, HD // 2), jnp.float32),
    )(ef, cut, w_lo, w_hi)


NB = 1000


def _tc_pre_body(z_ref, emb_ref, wm_ref, s_ref, m_ref):
    zb = z_ref[...]
    oh = (zb == lax.broadcasted_iota(jnp.int32, (NB, HD), 1)).astype(jnp.float32)
    s = jnp.dot(oh, emb_ref[...], preferred_element_type=jnp.float32, precision=lax.Precision.HIGHEST)
    s_ref[...] = s
    m_ref[...] = jnp.dot(s, wm_ref[...], preferred_element_type=jnp.float32, precision=lax.Precision.HIGHEST)


def _tc_pre(z2, emb_pad, wm0):
    return pl.pallas_call(
        _tc_pre_body,
        grid=(N // NB,),
        in_specs=[
            pl.BlockSpec((NB, 1), lambda j: (j, 0)),
            pl.BlockSpec((HD, HD), lambda j: (0, 0)),
            pl.BlockSpec((HD, HD), lambda j: (0, 0)),
        ],
        out_specs=[
            pl.BlockSpec((NB, HD), lambda j: (j, 0)),
            pl.BlockSpec((NB, HD), lambda j: (j, 0)),
        ],
        out_shape=[
            jax.ShapeDtypeStruct((N, HD), jnp.float32),
            jax.ShapeDtypeStruct((N, HD), jnp.float32),
        ],
    )(z2, emb_pad, wm0)


def _tc_upd_body(s_ref, a0_ref, a1_ref, wus_ref, wua_ref, b_ref, wm_ref,
                 s2_ref, m_ref):
    agg = a0_ref[0] + a1_ref[0]
    pre = (jnp.dot(s_ref[...], wus_ref[...], preferred_element_type=jnp.float32, precision=lax.Precision.HIGHEST)
           + jnp.dot(agg, wua_ref[...], preferred_element_type=jnp.float32, precision=lax.Precision.HIGHEST)
           + b_ref[...])
    s2 = s_ref[...] + _silu(pre)
    s2_ref[...] = s2
    m_ref[...] = jnp.dot(s2, wm_ref[...], preferred_element_type=jnp.float32, precision=lax.Precision.HIGHEST)


def _tc_upd(scalar, agg2, wus, wua, b, wm_next):
    return pl.pallas_call(
        _tc_upd_body,
        grid=(N // NB,),
        in_specs=[
            pl.BlockSpec((NB, HD), lambda j: (j, 0)),
            pl.BlockSpec((1, NB, HD), lambda j: (0, j, 0)),
            pl.BlockSpec((1, NB, HD), lambda j: (1, j, 0)),
            pl.BlockSpec((HD, HD), lambda j: (0, 0)),
            pl.BlockSpec((HD, HD), lambda j: (0, 0)),
            pl.BlockSpec((1, HD), lambda j: (0, 0)),
            pl.BlockSpec((HD, HD), lambda j: (0, 0)),
        ],
        out_specs=[
            pl.BlockSpec((NB, HD), lambda j: (j, 0)),
            pl.BlockSpec((NB, HD), lambda j: (j, 0)),
        ],
        out_shape=[
            jax.ShapeDtypeStruct((N, HD), jnp.float32),
            jax.ShapeDtypeStruct((N, HD), jnp.float32),
        ],
    )(scalar, agg2, agg2, wus, wua, b, wm_next)


def _tc_final_body(s_ref, a0_ref, a1_ref, wus_ref, wua_ref, b_ref, bi_ref,
                   wo_ref, acc_ref, pred_ref):
    j = pl.program_id(0)

    @pl.when(j == 0)
    def _():
        acc_ref[...] = jnp.zeros((NG, HD), jnp.float32)

    agg = a0_ref[0] + a1_ref[0]
    pre = (jnp.dot(s_ref[...], wus_ref[...], preferred_element_type=jnp.float32, precision=lax.Precision.HIGHEST)
           + jnp.dot(agg, wua_ref[...], preferred_element_type=jnp.float32, precision=lax.Precision.HIGHEST)
           + b_ref[...])
    s2 = s_ref[...] + _silu(pre)
    oh = (bi_ref[...] == lax.broadcasted_iota(jnp.int32, (NB, NG), 1)).astype(jnp.float32)
    acc_ref[...] += lax.dot_general(oh, s2, (((0,), (0,)), ((), ())),
                                    preferred_element_type=jnp.float32,
                                    precision=lax.Precision.HIGHEST)

    @pl.when(j == pl.num_programs(0) - 1)
    def _():
        graph = acc_ref[...] * C
        pred = jnp.sum(graph * wo_ref[...], axis=1, keepdims=True)
        pred_ref[...] = pred * Y_STD + Y_MEAN


def _tc_final(scalar, agg2, wus, wua, b, bi2, wo_t):
    return pl.pallas_call(
        _tc_final_body,
        grid=(N // NB,),
        in_specs=[
            pl.BlockSpec((NB, HD), lambda j: (j, 0)),
            pl.BlockSpec((1, NB, HD), lambda j: (0, j, 0)),
            pl.BlockSpec((1, NB, HD), lambda j: (1, j, 0)),
            pl.BlockSpec((HD, HD), lambda j: (0, 0)),
            pl.BlockSpec((HD, HD), lambda j: (0, 0)),
            pl.BlockSpec((1, HD), lambda j: (0, 0)),
            pl.BlockSpec((NB, 1), lambda j: (j, 0)),
            pl.BlockSpec((1, HD), lambda j: (0, 0)),
        ],
        out_specs=[
            pl.BlockSpec((NG, HD), lambda j: (0, 0)),
            pl.BlockSpec((NG, 1), lambda j: (0, 0)),
        ],
        out_shape=[
            jax.ShapeDtypeStruct((NG, HD), jnp.float32),
            jax.ShapeDtypeStruct((NG, 1), jnp.float32),
        ],
    )(scalar, agg2, agg2, wus, wua, b, bi2, wo_t)


def kernel(pos, z, x, edge_index, batch_index, subg_node_index,
           subg_node_center_index, subg_edge_index, subg_batch_index,
           subg_node_label, emb_table, W_ef, W_msg, W_upd, b_upd, W_out):
    src = edge_index[0].astype(jnp.int32).reshape(NW, NCH, K)
    dst = edge_index[1].astype(jnp.int32).reshape(NW, NCH, K)
    src2 = edge_index[0].astype(jnp.int32).reshape(NW, NCH2, K2)
    dst2 = edge_index[1].astype(jnp.int32).reshape(NW, NCH2, K2)
    zeros_nh = jnp.zeros((N, HD), jnp.float32)
    emb_pad = jnp.zeros((HD, HD), jnp.float32).at[:MAXZ].set(emb_table)

    pos16 = jnp.pad(pos.astype(jnp.float32), ((0, 0), (0, 13)))
    ps, pd = _sc_geom(pos16, src, dst)
    ef, cut = _tc_ef(ps, pd)
    gates = [_tc_gates(ef, cut, W_ef[l][:, _PERM_LO], W_ef[l][:, _PERM_HI])
             for l in range(LAYERS)]

    scalar, node_msg = _tc_pre(z.astype(jnp.int32).reshape(N, 1), emb_pad, W_msg[0])

    for l in range(LAYERS):
        agg2 = _sc_msg(node_msg, gates[l], src2, dst2, zeros_nh)
        wus = W_upd[l, :HD]
        wua = W_upd[l, HD:]
        b = b_upd[l].reshape(1, HD)
        if l < LAYERS - 1:
            scalar, node_msg = _tc_upd(scalar, agg2, wus, wua, b, W_msg[l + 1])
        else:
            _, pred = _tc_final(scalar, agg2, wus, wua, b,
                                batch_index.astype(jnp.int32).reshape(N, 1),
                                W_out.reshape(1, HD))
    return pred


# speedup vs baseline: 2.7219x; 1.4562x over previous
import functools

import numpy as np

import jax
import jax.numpy as jnp
from jax import lax
from jax.experimental import pallas as pl
from jax.experimental.pallas import tpu as pltpu
from jax.experimental.pallas import tpu_sc as plsc

N = 10000
E = 320000
NG = 64
HD = 128
EF = 16
MAXZ = 100
CUT = 10.0
RBOUND = 10.0
LAYERS = 4
C = 1.0
Y_STD = 1.0
Y_MEAN = 0.0

NC = 2
NS = 16
NW = NC * NS
EPT = E // NW
K = 80
NCH = EPT // K
K2 = 40
NCH2 = EPT // K2
SPT = N // NS

_mesh = plsc.VectorSubcoreMesh(core_axis_name="c", subcore_axis_name="s")
_sc_params = pltpu.CompilerParams(use_tc_tiling_on_sc=False)

_PERM_LO = np.concatenate([np.arange(32 * k, 32 * k + 16) for k in range(HD // 32)])
_PERM_HI = _PERM_LO + 16

_S16 = np.kron(np.eye(8, dtype=np.float32), np.ones((16, 16), np.float32))
_SX_BLOCK = np.zeros((16, 64), np.float32)
_SX_BLOCK[0, :] = 1.0
_SX = np.kron(np.eye(8, dtype=np.float32), _SX_BLOCK)


def _silu(x):
    return x * (1.0 / (1.0 + jnp.exp(-x)))


@functools.partial(
    pl.kernel,
    out_type=[
        jax.ShapeDtypeStruct((E, 16), jnp.float32),
        jax.ShapeDtypeStruct((E, 16), jnp.float32),
    ],
    mesh=_mesh,
    scratch_types=[
        pltpu.VMEM((NCH, K), jnp.int32),
        pltpu.VMEM((NCH, K), jnp.int32),
        pltpu.VMEM((K, 16), jnp.float32),
        pltpu.VMEM((K, 16), jnp.float32),
        pltpu.SemaphoreType.DMA,
        pltpu.SemaphoreType.DMA,
    ],
    compiler_params=_sc_params,
)
def _sc_geom(pos_hbm, src_hbm, dst_hbm, ps_hbm, pd_hbm,
             src_v, dst_v, ps_v, pd_v, sem_a, sem_b):
    cid = lax.axis_index("c")
    sid = lax.axis_index("s")
    wid = cid * NS + sid
    base = wid * EPT
    pltpu.sync_copy(src_hbm.at[wid], src_v)
    pltpu.sync_copy(dst_hbm.at[wid], dst_v)

    @pl.loop(0, NCH)
    def _chunk(j):
        ca = pltpu.async_copy(pos_hbm.at[src_v.at[j]], ps_v, sem_a)
        cb = pltpu.async_copy(pos_hbm.at[dst_v.at[j]], pd_v, sem_b)
        ca.wait()
        cb.wait()
        pltpu.sync_copy(ps_v, ps_hbm.at[pl.ds(base + j * K, K)])
        pltpu.sync_copy(pd_v, pd_hbm.at[pl.ds(base + j * K, K)])


@functools.partial(
    pl.kernel,
    out_type=jax.ShapeDtypeStruct((NC, N, HD), jnp.float32),
    mesh=_mesh,
    scratch_types=[
        pltpu.VMEM((NCH2, K2), jnp.int32),
        pltpu.VMEM((NCH2, K2), jnp.int32),
        pltpu.VMEM((K2, HD), jnp.float32),
        pltpu.VMEM((K2, HD), jnp.float32),
        pltpu.VMEM((K2, HD // 2), jnp.float32),
        pltpu.VMEM((K2, HD // 2), jnp.float32),
        pltpu.VMEM_SHARED((N, HD), jnp.float32),
        pltpu.SemaphoreType.DMA,
        pltpu.SemaphoreType.DMA,
        pltpu.SemaphoreType.DMA,
        pltpu.SemaphoreType.DMA,
    ],
    compiler_params=_sc_params,
)
def _sc_msg(msg_hbm, gate_hbm, src_hbm, dst_hbm, zeros_hbm, agg2_hbm,
            src_v, dst_v, rows_v0, rows_v1, gate_v0, gate_v1, agg_sh,
            sem_r0, sem_r1, sem_g0, sem_g1):
    rows = (rows_v0, rows_v1)
    gbuf = (gate_v0, gate_v1)
    sem_r = (sem_r0, sem_r1)
    sem_g = (sem_g0, sem_g1)
    cid = lax.axis_index("c")
    sid = lax.axis_index("s")
    wid = cid * NS + sid
    base = wid * EPT
    pltpu.sync_copy(zeros_hbm.at[pl.ds(sid * SPT, SPT)],
                    agg_sh.at[pl.ds(sid * SPT, SPT)])
    pltpu.sync_copy(src_hbm.at[wid], src_v)
    pltpu.sync_copy(dst_hbm.at[wid], dst_v)
    plsc.subcore_barrier()

    def issue(j, b):
        pltpu.async_copy(gate_hbm.at[pl.ds(base + j * K2, K2)], gbuf[b], sem_g[b])
        pltpu.async_copy(msg_hbm.at[src_v.at[j]], rows[b], sem_r[b])

    def work(j, b):
        pltpu.make_async_copy(gate_hbm.at[pl.ds(base + j * K2, K2)],
                              gbuf[b], sem_g[b]).wait()
        pltpu.make_async_copy(msg_hbm.at[src_v.at[j]], rows[b], sem_r[b]).wait()

        @pl.loop(0, K2)
        def _row(i):
            for k in range(HD // 32):
                v = lax.bitcast_convert_type(gbuf[b][i, pl.ds(k * 16, 16)],
                                             jnp.uint32)
                ga = lax.bitcast_convert_type(v << 16, jnp.float32)
                gb2 = lax.bitcast_convert_type(v & jnp.uint32(0xFFFF0000),
                                               jnp.float32)
                sl_a = pl.ds(k * 32, 16)
                sl_b = pl.ds(k * 32 + 16, 16)
                rows[b][i, sl_a] = rows[b][i, sl_a] * ga
                rows[b][i, sl_b] = rows[b][i, sl_b] * gb2

        pltpu.sync_copy(rows[b], agg_sh.at[dst_v.at[j]], add=True)

    issue(0, 0)
    issue(1, 1)

    @pl.loop(0, (NCH2 - 2) // 2)
    def _pair(p):
        j0 = p * 2
        work(j0, 0)
        issue(j0 + 2, 0)
        work(j0 + 1, 1)
        issue(j0 + 3, 1)

    work(NCH2 - 2, 0)
    work(NCH2 - 1, 1)

    plsc.subcore_barrier()
    pltpu.sync_copy(agg_sh.at[pl.ds(sid * SPT, SPT)],
                    agg2_hbm.at[cid, pl.ds(sid * SPT, SPT)])


EB = 8000


BG = E // 8
EBG = 1000


def _tc_ef_body(ps_ref, pd_ref, s16_ref, ef_ref, cut_ref):
    diff = ps_ref[...] - pd_ref[...]
    sq = diff * diff
    d2 = jnp.dot(sq, s16_ref[...], preferred_element_type=jnp.float32,
                 precision=lax.Precision.HIGHEST)
    dist = jnp.sqrt(d2 + 1e-12)
    step = RBOUND / (EF - 1)
    centers = (lax.broadcasted_iota(jnp.int32, (EBG, HD), 1) % EF
               ).astype(jnp.float32) * step
    gamma = (EF / RBOUND) ** 2
    ef_ref[...] = jnp.exp(-gamma * (dist - centers) ** 2)
    cut = 0.5 * (jnp.cos(jnp.pi * jnp.clip(dist / CUT, 0.0, 1.0)) + 1.0)
    cut_ref[...] = cut * (dist < CUT).astype(jnp.float32)


def _tc_ef(ps2, pd2, s16):
    return pl.pallas_call(
        _tc_ef_body,
        grid=(BG // EBG,),
        in_specs=[
            pl.BlockSpec((EBG, HD), lambda j: (j, 0)),
            pl.BlockSpec((EBG, HD), lambda j: (j, 0)),
            pl.BlockSpec((HD, HD), lambda j: (0, 0)),
        ],
        out_specs=[
            pl.BlockSpec((EBG, HD), lambda j: (j, 0)),
            pl.BlockSpec((EBG, HD), lambda j: (j, 0)),
        ],
        out_shape=[
            jax.ShapeDtypeStruct((BG, HD), jnp.float32),
            jax.ShapeDtypeStruct((BG, HD), jnp.float32),
        ],
    )(ps2, pd2, s16)


def _tc_gates_body(ef_ref, cut_ref, wlo_ref, whi_ref, sx_ref, out_ref):
    ef = ef_ref[...]
    cut512 = jnp.dot(cut_ref[...], sx_ref[...],
                     preferred_element_type=jnp.float32,
                     precision=lax.Precision.HIGHEST)
    g_lo = _silu(jnp.dot(ef, wlo_ref[...],
                         preferred_element_type=jnp.float32)) * cut512
    g_hi = _silu(jnp.dot(ef, whi_ref[...],
                         preferred_element_type=jnp.float32)) * cut512
    lo = lax.bitcast_convert_type(g_lo.astype(jnp.bfloat16),
                                  jnp.uint16).astype(jnp.uint32)
    hi = lax.bitcast_convert_type(g_hi.astype(jnp.bfloat16),
                                  jnp.uint16).astype(jnp.uint32)
    out_ref[...] = lax.bitcast_convert_type(lo | (hi << 16), jnp.float32)


def _tc_gates(ef, cut, w_lo_big, w_hi_big, sx):
    return pl.pallas_call(
        _tc_gates_body,
        grid=(BG // EBG,),
        in_specs=[
            pl.BlockSpec((EBG, HD), lambda j: (j, 0)),
            pl.BlockSpec((EBG, HD), lambda j: (j, 0)),
            pl.BlockSpec((HD, 512), lambda j: (0, 0)),
            pl.BlockSpec((HD, 512), lambda j: (0, 0)),
            pl.BlockSpec((HD, 512), lambda j: (0, 0)),
        ],
        out_specs=pl.BlockSpec((EBG, 512), lambda j: (j, 0)),
        out_shape=jax.ShapeDtypeStruct((BG, 512), jnp.float32),
    )(ef, cut, w_lo_big, w_hi_big, sx)


NB = 1000


def _tc_pre_body(z_ref, emb_ref, wm_ref, s_ref, m_ref):
    zb = z_ref[...]
    oh = (zb == lax.broadcasted_iota(jnp.int32, (NB, HD), 1)).astype(jnp.float32)
    s = jnp.dot(oh, emb_ref[...], preferred_element_type=jnp.float32, precision=lax.Precision.HIGHEST)
    s_ref[...] = s
    m_ref[...] = jnp.dot(s, wm_ref[...], preferred_element_type=jnp.float32, precision=lax.Precision.HIGHEST)


def _tc_pre(z2, emb_pad, wm0):
    return pl.pallas_call(
        _tc_pre_body,
        grid=(N // NB,),
        in_specs=[
            pl.BlockSpec((NB, 1), lambda j: (j, 0)),
            pl.BlockSpec((HD, HD), lambda j: (0, 0)),
            pl.BlockSpec((HD, HD), lambda j: (0, 0)),
        ],
        out_specs=[
            pl.BlockSpec((NB, HD), lambda j: (j, 0)),
            pl.BlockSpec((NB, HD), lambda j: (j, 0)),
        ],
        out_shape=[
            jax.ShapeDtypeStruct((N, HD), jnp.float32),
            jax.ShapeDtypeStruct((N, HD), jnp.float32),
        ],
    )(z2, emb_pad, wm0)


def _tc_upd_body(s_ref, a0_ref, a1_ref, wus_ref, wua_ref, b_ref, wm_ref,
                 s2_ref, m_ref):
    agg = a0_ref[0] + a1_ref[0]
    pre = (jnp.dot(s_ref[...], wus_ref[...], preferred_element_type=jnp.float32, precision=lax.Precision.HIGHEST)
           + jnp.dot(agg, wua_ref[...], preferred_element_type=jnp.float32, precision=lax.Precision.HIGHEST)
           + b_ref[...])
    s2 = s_ref[...] + _silu(pre)
    s2_ref[...] = s2
    m_ref[...] = jnp.dot(s2, wm_ref[...], preferred_element_type=jnp.float32, precision=lax.Precision.HIGHEST)


def _tc_upd(scalar, agg2, wus, wua, b, wm_next):
    return pl.pallas_call(
        _tc_upd_body,
        grid=(N // NB,),
        in_specs=[
            pl.BlockSpec((NB, HD), lambda j: (j, 0)),
            pl.BlockSpec((1, NB, HD), lambda j: (0, j, 0)),
            pl.BlockSpec((1, NB, HD), lambda j: (1, j, 0)),
            pl.BlockSpec((HD, HD), lambda j: (0, 0)),
            pl.BlockSpec((HD, HD), lambda j: (0, 0)),
            pl.BlockSpec((1, HD), lambda j: (0, 0)),
            pl.BlockSpec((HD, HD), lambda j: (0, 0)),
        ],
        out_specs=[
            pl.BlockSpec((NB, HD), lambda j: (j, 0)),
            pl.BlockSpec((NB, HD), lambda j: (j, 0)),
        ],
        out_shape=[
            jax.ShapeDtypeStruct((N, HD), jnp.float32),
            jax.ShapeDtypeStruct((N, HD), jnp.float32),
        ],
    )(scalar, agg2, agg2, wus, wua, b, wm_next)


def _tc_final_body(s_ref, a0_ref, a1_ref, wus_ref, wua_ref, b_ref, bi_ref,
                   wo_ref, acc_ref, pred_ref):
    j = pl.program_id(0)

    @pl.when(j == 0)
    def _():
        acc_ref[...] = jnp.zeros((NG, HD), jnp.float32)

    agg = a0_ref[0] + a1_ref[0]
    pre = (jnp.dot(s_ref[...], wus_ref[...], preferred_element_type=jnp.float32, precision=lax.Precision.HIGHEST)
           + jnp.dot(agg, wua_ref[...], preferred_element_type=jnp.float32, precision=lax.Precision.HIGHEST)
           + b_ref[...])
    s2 = s_ref[...] + _silu(pre)
    oh = (bi_ref[...] == lax.broadcasted_iota(jnp.int32, (NB, NG), 1)).astype(jnp.float32)
    acc_ref[...] += lax.dot_general(oh, s2, (((0,), (0,)), ((), ())),
                                    preferred_element_type=jnp.float32,
                                    precision=lax.Precision.HIGHEST)

    @pl.when(j == pl.num_programs(0) - 1)
    def _():
        graph = acc_ref[...] * C
        pred = jnp.sum(graph * wo_ref[...], axis=1, keepdims=True)
        pred_ref[...] = pred * Y_STD + Y_MEAN


def _tc_final(scalar, agg2, wus, wua, b, bi2, wo_t):
    return pl.pallas_call(
        _tc_final_body,
        grid=(N // NB,),
        in_specs=[
            pl.BlockSpec((NB, HD), lambda j: (j, 0)),
            pl.BlockSpec((1, NB, HD), lambda j: (0, j, 0)),
            pl.BlockSpec((1, NB, HD), lambda j: (1, j, 0)),
            pl.BlockSpec((HD, HD), lambda j: (0, 0)),
            pl.BlockSpec((HD, HD), lambda j: (0, 0)),
            pl.BlockSpec((1, HD), lambda j: (0, 0)),
            pl.BlockSpec((NB, 1), lambda j: (j, 0)),
            pl.BlockSpec((1, HD), lambda j: (0, 0)),
        ],
        out_specs=[
            pl.BlockSpec((NG, HD), lambda j: (0, 0)),
            pl.BlockSpec((NG, 1), lambda j: (0, 0)),
        ],
        out_shape=[
            jax.ShapeDtypeStruct((NG, HD), jnp.float32),
            jax.ShapeDtypeStruct((NG, 1), jnp.float32),
        ],
    )(scalar, agg2, agg2, wus, wua, b, bi2, wo_t)


def kernel(pos, z, x, edge_index, batch_index, subg_node_index,
           subg_node_center_index, subg_edge_index, subg_batch_index,
           subg_node_label, emb_table, W_ef, W_msg, W_upd, b_upd, W_out):
    src = edge_index[0].astype(jnp.int32).reshape(NW, NCH, K)
    dst = edge_index[1].astype(jnp.int32).reshape(NW, NCH, K)
    src2 = edge_index[0].astype(jnp.int32).reshape(NW, NCH2, K2)
    dst2 = edge_index[1].astype(jnp.int32).reshape(NW, NCH2, K2)
    zeros_nh = jnp.zeros((N, HD), jnp.float32)
    emb_pad = jnp.zeros((HD, HD), jnp.float32).at[:MAXZ].set(emb_table)

    pos16 = jnp.pad(pos.astype(jnp.float32), ((0, 0), (0, 13)))
    ps, pd = _sc_geom(pos16, src, dst)
    s16 = jnp.asarray(_S16)
    sx = jnp.asarray(_SX)
    eye8 = jnp.eye(8, dtype=jnp.float32)
    ef, cut = _tc_ef(ps.reshape(BG, HD), pd.reshape(BG, HD), s16)
    gates = [
        _tc_gates(ef, cut,
                  jnp.kron(eye8, W_ef[l][:, _PERM_LO]),
                  jnp.kron(eye8, W_ef[l][:, _PERM_HI]), sx).reshape(E, HD // 2)
        for l in range(LAYERS)
    ]

    scalar, node_msg = _tc_pre(z.astype(jnp.int32).reshape(N, 1), emb_pad, W_msg[0])

    for l in range(LAYERS):
        agg2 = _sc_msg(node_msg, gates[l], src2, dst2, zeros_nh)
        wus = W_upd[l, :HD]
        wua = W_upd[l, HD:]
        b = b_upd[l].reshape(1, HD)
        if l < LAYERS - 1:
            scalar, node_msg = _tc_upd(scalar, agg2, wus, wua, b, W_msg[l + 1])
        else:
            _, pred = _tc_final(scalar, agg2, wus, wua, b,
                                batch_index.astype(jnp.int32).reshape(N, 1),
                                W_out.reshape(1, HD))
    return pred
